# Initial kernel scaffold; baseline (speedup 1.0000x reference)
import jax
import jax.numpy as jnp
from jax.experimental import pallas as pl


def kernel(x, edge_index, W, b):
    raise NotImplementedError("write your pallas kernel here")


# speedup vs baseline: 4.8546x; 4.8546x over previous
import functools

import jax
import jax.numpy as jnp
from jax import lax
from jax.experimental import pallas as pl
from jax.experimental.pallas import tpu as pltpu
from jax.experimental.pallas import tpu_sc as plsc

N_NODES = 10000
N_EDGES = 160000
D_IN = 256
D_OUT = 256
DH = D_IN // 2

NC = 2
NS = 16
CH = 80
G = 25
NG = N_EDGES // (G * CH * NS)
NPT = 632
LAST_START = N_NODES - NPT


def _sc_aggregate(x2, src2, dst2, zrows):
    mesh = plsc.VectorSubcoreMesh(core_axis_name="c", subcore_axis_name="s")

    @functools.partial(
        pl.kernel,
        out_type=jax.ShapeDtypeStruct((NC, N_NODES, DH), jnp.float32),
        mesh=mesh,
        scratch_types=[
            pltpu.VMEM_SHARED((N_NODES, DH), jnp.float32),
            pltpu.VMEM((G, CH), jnp.int32),
            pltpu.VMEM((G, CH), jnp.int32),
            pltpu.VMEM((CH, DH), jnp.float32),
            pltpu.SemaphoreType.DMA,
        ],
    )
    def body(x2_hbm, src_hbm, dst_hbm, z_hbm, out_hbm, acc, sbuf, dbuf, rows, sem):
        c = lax.axis_index("c")
        s = lax.axis_index("s")

        start = pl.multiple_of(jnp.minimum(s * NPT, LAST_START), 8)
        pltpu.sync_copy(z_hbm, acc.at[pl.ds(start, NPT)])
        plsc.subcore_barrier()

        def outer(g, _):
            k = s * NG + g
            pltpu.sync_copy(src_hbm.at[k], sbuf)
            pltpu.sync_copy(dst_hbm.at[k], dbuf)

            def inner(j, _):
                pltpu.async_copy(x2_hbm.at[c].at[sbuf.at[j]], rows, sem).wait()
                pltpu.sync_copy(rows, acc.at[dbuf.at[j]], add=True)
                return 0

            lax.fori_loop(0, G, inner, 0, unroll=False)
            return 0

        lax.fori_loop(0, NG, outer, 0, unroll=False)
        plsc.subcore_barrier()

        pltpu.sync_copy(
            acc.at[pl.ds(start, NPT)],
            out_hbm.at[c].at[pl.ds(start, NPT)],
        )

    return body(x2, src2, dst2, zrows)


def _tc_linear(agg2, w0t, w1t, b2):
    BN = 400

    def mm(a_ref, w0_ref, w1_ref, b_ref, o_ref):
        acc = jnp.dot(a_ref[0], w0_ref[...], preferred_element_type=jnp.float32)
        acc = acc + jnp.dot(a_ref[1], w1_ref[...], preferred_element_type=jnp.float32)
        o_ref[...] = acc + b_ref[...]

    return pl.pallas_call(
        mm,
        grid=(N_NODES // BN,),
        in_specs=[
            pl.BlockSpec((2, BN, DH), lambda i: (0, i, 0)),
            pl.BlockSpec((DH, D_OUT), lambda i: (0, 0)),
            pl.BlockSpec((DH, D_OUT), lambda i: (0, 0)),
            pl.BlockSpec((1, D_OUT), lambda i: (0, 0)),
        ],
        out_specs=pl.BlockSpec((BN, D_OUT), lambda i: (i, 0)),
        out_shape=jax.ShapeDtypeStruct((N_NODES, D_OUT), jnp.float32),
    )(agg2, w0t, w1t, b2)


def kernel(x, edge_index, W, b):
    src2 = edge_index[0].astype(jnp.int32).reshape(NS * NG, G, CH)
    dst2 = edge_index[1].astype(jnp.int32).reshape(NS * NG, G, CH)
    x2 = jnp.stack([x[:, :DH], x[:, DH:]], axis=0)
    zrows = jnp.zeros((NPT, DH), jnp.float32)
    w0t = W[:, :DH].T
    w1t = W[:, DH:].T
    b2 = b.reshape(1, D_OUT)
    agg2 = _sc_aggregate(x2, src2, dst2, zrows)
    return _tc_linear(agg2, w0t, w1t, b2)


# speedup vs baseline: 7.2009x; 1.4833x over previous
import functools

import jax
import jax.numpy as jnp
from jax import lax
from jax.experimental import pallas as pl
from jax.experimental.pallas import tpu as pltpu
from jax.experimental.pallas import tpu_sc as plsc

N_NODES = 10000
N_EDGES = 160000
D_IN = 256
D_OUT = 256
DH = D_IN // 2

NC = 2
NS = 16
CH = 80
G = 25
NG = N_EDGES // (G * CH * NS)
T = NG * G
NPT = 632
LAST_START = N_NODES - NPT


def _sc_aggregate(x2, src2, dst2, zrows):
    mesh = plsc.VectorSubcoreMesh(core_axis_name="c", subcore_axis_name="s")

    @functools.partial(
        pl.kernel,
        out_type=jax.ShapeDtypeStruct((NC, N_NODES, DH), jnp.float32),
        mesh=mesh,
        scratch_types=[
            pltpu.VMEM_SHARED((N_NODES, DH), jnp.float32),
            pltpu.VMEM((2, G, CH), jnp.int32),
            pltpu.VMEM((2, G, CH), jnp.int32),
            pltpu.VMEM((2, CH, DH), jnp.float32),
            pltpu.SemaphoreType.DMA,
            pltpu.SemaphoreType.DMA,
        ],
    )
    def body(x2_hbm, src_hbm, dst_hbm, z_hbm, out_hbm, acc, sbuf, dbuf, rows,
             sem_g, sem_s):
        c = lax.axis_index("c")
        s = lax.axis_index("s")

        def gather(j):
            return pltpu.make_async_copy(
                x2_hbm.at[c].at[sbuf.at[(j // G) % 2, j % G]],
                rows.at[j % 2], sem_g)

        def scatter(j):
            return pltpu.make_async_copy(
                rows.at[j % 2],
                acc.at[dbuf.at[(j // G) % 2, j % G]], sem_s)

        def stage(g):
            pltpu.sync_copy(src_hbm.at[s * NG + g], sbuf.at[g % 2])
            pltpu.sync_copy(dst_hbm.at[s * NG + g], dbuf.at[g % 2])

        start = pl.multiple_of(jnp.minimum(s * NPT, LAST_START), 8)
        pltpu.sync_copy(z_hbm, acc.at[pl.ds(start, NPT)])
        stage(0)
        plsc.subcore_barrier()
        gather(0).start()

        def step(j, _):
            @pl.when(j >= 1)
            def _():
                scatter(j - 1).wait()

            @pl.when(j + 1 < T)
            def _():
                @pl.when((j + 1) % G == 0)
                def _():
                    stage((j + 1) // G)
                gather(j + 1).start()

            gather(j).wait()
            pltpu.async_copy(rows.at[j % 2],
                             acc.at[dbuf.at[(j // G) % 2, j % G]],
                             sem_s, add=True)
            return 0

        lax.fori_loop(0, T, step, 0, unroll=False)
        scatter(T - 1).wait()
        plsc.subcore_barrier()

        pltpu.sync_copy(
            acc.at[pl.ds(start, NPT)],
            out_hbm.at[c].at[pl.ds(start, NPT)],
        )

    return body(x2, src2, dst2, zrows)


def _tc_linear(agg2, w0t, w1t, b2):
    BN = 400

    def mm(a_ref, w0_ref, w1_ref, b_ref, o_ref):
        acc = jnp.dot(a_ref[0], w0_ref[...], preferred_element_type=jnp.float32)
        acc = acc + jnp.dot(a_ref[1], w1_ref[...], preferred_element_type=jnp.float32)
        o_ref[...] = acc + b_ref[...]

    return pl.pallas_call(
        mm,
        grid=(N_NODES // BN,),
        in_specs=[
            pl.BlockSpec((2, BN, DH), lambda i: (0, i, 0)),
            pl.BlockSpec((DH, D_OUT), lambda i: (0, 0)),
            pl.BlockSpec((DH, D_OUT), lambda i: (0, 0)),
            pl.BlockSpec((1, D_OUT), lambda i: (0, 0)),
        ],
        out_specs=pl.BlockSpec((BN, D_OUT), lambda i: (i, 0)),
        out_shape=jax.ShapeDtypeStruct((N_NODES, D_OUT), jnp.float32),
    )(agg2, w0t, w1t, b2)


def kernel(x, edge_index, W, b):
    src2 = edge_index[0].astype(jnp.int32).reshape(NS * NG, G, CH)
    dst2 = edge_index[1].astype(jnp.int32).reshape(NS * NG, G, CH)
    x2 = jnp.stack([x[:, :DH], x[:, DH:]], axis=0)
    zrows = jnp.zeros((NPT, DH), jnp.float32)
    w0t = W[:, :DH].T
    w1t = W[:, DH:].T
    b2 = b.reshape(1, D_OUT)
    agg2 = _sc_aggregate(x2, src2, dst2, zrows)
    return _tc_linear(agg2, w0t, w1t, b2)


# speedup vs baseline: 8.4494x; 1.1734x over previous
import functools

import jax
import jax.numpy as jnp
from jax import lax
from jax.experimental import pallas as pl
from jax.experimental.pallas import tpu as pltpu
from jax.experimental.pallas import tpu_sc as plsc

N_NODES = 10000
N_EDGES = 160000
D_IN = 256
D_OUT = 256
DH = D_IN // 2

NC = 2
NS = 16
CH = 80
G = 25
NG = N_EDGES // (G * CH * NS)
T = NG * G
R = 3
NPT = 632
LAST_START = N_NODES - NPT


def _sc_aggregate(x2, src2, dst2, zrows):
    mesh = plsc.VectorSubcoreMesh(core_axis_name="c", subcore_axis_name="s")

    @functools.partial(
        pl.kernel,
        out_type=jax.ShapeDtypeStruct((NC, N_NODES, DH), jnp.float32),
        mesh=mesh,
        scratch_types=[
            pltpu.VMEM_SHARED((N_NODES, DH), jnp.float32),
            pltpu.VMEM((2, G, CH), jnp.int32),
            pltpu.VMEM((2, G, CH), jnp.int32),
            pltpu.VMEM((R, CH, DH), jnp.float32),
            pltpu.SemaphoreType.DMA,
            pltpu.SemaphoreType.DMA,
        ],
    )
    def body(x2_hbm, src_hbm, dst_hbm, z_hbm, out_hbm, acc, sbuf, dbuf, rows,
             sem_g, sem_s):
        c = lax.axis_index("c")
        s = lax.axis_index("s")

        def gather(j):
            return pltpu.make_async_copy(
                x2_hbm.at[c].at[sbuf.at[(j // G) % 2, j % G]],
                rows.at[j % R], sem_g)

        def scatter(j):
            return pltpu.make_async_copy(
                rows.at[j % R],
                acc.at[dbuf.at[(j // G) % 2, j % G]], sem_s)

        def stage(g):
            pltpu.sync_copy(src_hbm.at[s * NG + g], sbuf.at[g % 2])
            pltpu.sync_copy(dst_hbm.at[s * NG + g], dbuf.at[g % 2])

        start = pl.multiple_of(jnp.minimum(s * NPT, LAST_START), 8)
        pltpu.sync_copy(z_hbm, acc.at[pl.ds(start, NPT)])
        stage(0)
        plsc.subcore_barrier()
        for j0 in range(R - 1):
            gather(j0).start()

        def step(j, _):
            @pl.when(j >= R - 1)
            def _():
                scatter(j - (R - 1)).wait()

            @pl.when(j + R - 1 < T)
            def _():
                jn = j + R - 1

                @pl.when(jn % G == 0)
                def _():
                    stage(jn // G)
                gather(jn).start()

            gather(j).wait()
            pltpu.async_copy(rows.at[j % R],
                             acc.at[dbuf.at[(j // G) % 2, j % G]],
                             sem_s, add=True)
            return 0

        lax.fori_loop(0, T, step, 0, unroll=False)
        for j0 in range(T - (R - 1), T):
            scatter(j0).wait()
        plsc.subcore_barrier()

        pltpu.sync_copy(
            acc.at[pl.ds(start, NPT)],
            out_hbm.at[c].at[pl.ds(start, NPT)],
        )

    return body(x2, src2, dst2, zrows)


def _tc_linear(agg2, w0t, w1t, b2):
    BN = 400

    def mm(a_ref, w0_ref, w1_ref, b_ref, o_ref):
        acc = jnp.dot(a_ref[0], w0_ref[...], preferred_element_type=jnp.float32)
        acc = acc + jnp.dot(a_ref[1], w1_ref[...], preferred_element_type=jnp.float32)
        o_ref[...] = acc + b_ref[...]

    return pl.pallas_call(
        mm,
        grid=(N_NODES // BN,),
        in_specs=[
            pl.BlockSpec((2, BN, DH), lambda i: (0, i, 0)),
            pl.BlockSpec((DH, D_OUT), lambda i: (0, 0)),
            pl.BlockSpec((DH, D_OUT), lambda i: (0, 0)),
            pl.BlockSpec((1, D_OUT), lambda i: (0, 0)),
        ],
        out_specs=pl.BlockSpec((BN, D_OUT), lambda i: (i, 0)),
        out_shape=jax.ShapeDtypeStruct((N_NODES, D_OUT), jnp.float32),
    )(agg2, w0t, w1t, b2)


def kernel(x, edge_index, W, b):
    src2 = edge_index[0].astype(jnp.int32).reshape(NS * NG, G, CH)
    dst2 = edge_index[1].astype(jnp.int32).reshape(NS * NG, G, CH)
    x2 = jnp.stack([x[:, :DH], x[:, DH:]], axis=0)
    zrows = jnp.zeros((NPT, DH), jnp.float32)
    w0t = W[:, :DH].T
    w1t = W[:, DH:].T
    b2 = b.reshape(1, D_OUT)
    agg2 = _sc_aggregate(x2, src2, dst2, zrows)
    return _tc_linear(agg2, w0t, w1t, b2)


# speedup vs baseline: 8.8793x; 1.0509x over previous
import functools

import jax
import jax.numpy as jnp
from jax import lax
from jax.experimental import pallas as pl
from jax.experimental.pallas import tpu as pltpu
from jax.experimental.pallas import tpu_sc as plsc

N_NODES = 10000
N_EDGES = 160000
D_IN = 256
D_OUT = 256
DH = D_IN // 2

NC = 2
NS = 16
CH = 80
G = 25
NG = N_EDGES // (G * CH * NS)
T = NG * G
R = 3
NPT = 632
LAST_START = N_NODES - NPT


def _sc_aggregate(x2, src2, dst2, zrows):
    mesh = plsc.VectorSubcoreMesh(core_axis_name="c", subcore_axis_name="s")

    @functools.partial(
        pl.kernel,
        out_type=jax.ShapeDtypeStruct((NC, N_NODES, DH), jnp.float32),
        mesh=mesh,
        scratch_types=[
            pltpu.VMEM_SHARED((N_NODES, DH), jnp.float32),
            pltpu.VMEM((2, G, CH), jnp.int32),
            pltpu.VMEM((2, G, CH), jnp.int32),
            pltpu.VMEM((R, CH, DH), jnp.float32),
            pltpu.SemaphoreType.DMA,
            pltpu.SemaphoreType.DMA,
        ],
    )
    def body(x2_hbm, src_hbm, dst_hbm, z_hbm, out_hbm, acc, sbuf, dbuf, rows,
             sem_g, sem_s):
        c = lax.axis_index("c")
        s = lax.axis_index("s")

        def gather(j):
            return pltpu.make_async_copy(
                x2_hbm.at[sbuf.at[(j // G) % 2, j % G], pl.ds(c * DH, DH)],
                rows.at[j % R], sem_g)

        def scatter(j):
            return pltpu.make_async_copy(
                rows.at[j % R],
                acc.at[dbuf.at[(j // G) % 2, j % G]], sem_s)

        def stage(g):
            pltpu.sync_copy(src_hbm.at[s * NG + g], sbuf.at[g % 2])
            pltpu.sync_copy(dst_hbm.at[s * NG + g], dbuf.at[g % 2])

        start = pl.multiple_of(jnp.minimum(s * NPT, LAST_START), 8)
        pltpu.sync_copy(z_hbm, acc.at[pl.ds(start, NPT)])
        stage(0)
        plsc.subcore_barrier()
        for j0 in range(R - 1):
            gather(j0).start()

        def step(j, _):
            @pl.when(j >= R - 1)
            def _():
                scatter(j - (R - 1)).wait()

            @pl.when(j + R - 1 < T)
            def _():
                jn = j + R - 1

                @pl.when(jn % G == 0)
                def _():
                    stage(jn // G)
                gather(jn).start()

            gather(j).wait()
            pltpu.async_copy(rows.at[j % R],
                             acc.at[dbuf.at[(j // G) % 2, j % G]],
                             sem_s, add=True)
            return 0

        lax.fori_loop(0, T, step, 0, unroll=False)
        for j0 in range(T - (R - 1), T):
            scatter(j0).wait()
        plsc.subcore_barrier()

        pltpu.sync_copy(
            acc.at[pl.ds(start, NPT)],
            out_hbm.at[c].at[pl.ds(start, NPT)],
        )

    return body(x2, src2, dst2, zrows)


def _tc_linear(agg2, w0t, w1t, b2):
    BN = 400

    def mm(a_ref, w0_ref, w1_ref, b_ref, o_ref):
        acc = jnp.dot(a_ref[0], w0_ref[...], preferred_element_type=jnp.float32)
        acc = acc + jnp.dot(a_ref[1], w1_ref[...], preferred_element_type=jnp.float32)
        o_ref[...] = acc + b_ref[...]

    return pl.pallas_call(
        mm,
        grid=(N_NODES // BN,),
        in_specs=[
            pl.BlockSpec((2, BN, DH), lambda i: (0, i, 0)),
            pl.BlockSpec((DH, D_OUT), lambda i: (0, 0)),
            pl.BlockSpec((DH, D_OUT), lambda i: (0, 0)),
            pl.BlockSpec((1, D_OUT), lambda i: (0, 0)),
        ],
        out_specs=pl.BlockSpec((BN, D_OUT), lambda i: (i, 0)),
        out_shape=jax.ShapeDtypeStruct((N_NODES, D_OUT), jnp.float32),
    )(agg2, w0t, w1t, b2)


def kernel(x, edge_index, W, b):
    src2 = edge_index[0].astype(jnp.int32).reshape(NS * NG, G, CH)
    dst2 = edge_index[1].astype(jnp.int32).reshape(NS * NG, G, CH)
    zrows = jnp.zeros((NPT, DH), jnp.float32)
    w0t = W[:, :DH].T
    w1t = W[:, DH:].T
    b2 = b.reshape(1, D_OUT)
    agg2 = _sc_aggregate(x, src2, dst2, zrows)
    return _tc_linear(agg2, w0t, w1t, b2)


# speedup vs baseline: 9.5015x; 1.0701x over previous
import functools

import jax
import jax.numpy as jnp
from jax import lax
from jax.experimental import pallas as pl
from jax.experimental.pallas import tpu as pltpu
from jax.experimental.pallas import tpu_sc as plsc

N_NODES = 10000
N_EDGES = 160000
D_IN = 256
D_OUT = 256
DH = D_IN // 2

NC = 2
NS = 16
CH = 80
G = 25
NG = N_EDGES // (G * CH * NS)
T = NG * G
R = 3
NPT = 632
LAST_START = N_NODES - NPT


def _sc_aggregate(x2, src2, dst2, zrows):
    mesh = plsc.VectorSubcoreMesh(core_axis_name="c", subcore_axis_name="s")

    @functools.partial(
        pl.kernel,
        out_type=jax.ShapeDtypeStruct((NC, N_NODES, DH), jnp.float32),
        mesh=mesh,
        scratch_types=[
            pltpu.VMEM_SHARED((N_NODES, DH), jnp.float32),
            pltpu.VMEM((2, G, CH), jnp.int32),
            pltpu.VMEM((2, G, CH), jnp.int32),
            pltpu.VMEM((R, CH, DH), jnp.float32),
            pltpu.SemaphoreType.DMA,
            pltpu.SemaphoreType.DMA,
        ],
    )
    def body(x2_hbm, src_hbm, dst_hbm, z_hbm, out_hbm, acc, sbuf, dbuf, rows,
             sem_g, sem_s):
        c = lax.axis_index("c")
        s = lax.axis_index("s")

        def gather(j):
            return pltpu.make_async_copy(
                x2_hbm.at[sbuf.at[(j // G) % 2, j % G], pl.ds(c * DH, DH)],
                rows.at[j % R], sem_g)

        def scatter(j):
            return pltpu.make_async_copy(
                rows.at[j % R],
                acc.at[dbuf.at[(j // G) % 2, j % G]], sem_s)

        def stage(g):
            pltpu.sync_copy(src_hbm.at[s * NG + g], sbuf.at[g % 2])
            pltpu.sync_copy(dst_hbm.at[s * NG + g], dbuf.at[g % 2])

        stage(0)
        for j0 in range(R - 1):
            gather(j0).start()
        start = pl.multiple_of(jnp.minimum(s * NPT, LAST_START), 8)
        pltpu.sync_copy(z_hbm, acc.at[pl.ds(start, NPT)])
        plsc.subcore_barrier()

        def step(j, _):
            @pl.when(j >= R - 1)
            def _():
                scatter(j - (R - 1)).wait()

            @pl.when(j + R - 1 < T)
            def _():
                jn = j + R - 1

                @pl.when(jn % G == 0)
                def _():
                    stage(jn // G)
                gather(jn).start()

            gather(j).wait()
            pltpu.async_copy(rows.at[j % R],
                             acc.at[dbuf.at[(j // G) % 2, j % G]],
                             sem_s, add=True)
            return 0

        lax.fori_loop(0, T, step, 0, unroll=False)
        for j0 in range(T - (R - 1), T):
            scatter(j0).wait()
        plsc.subcore_barrier()

        pltpu.sync_copy(
            acc.at[pl.ds(start, NPT)],
            out_hbm.at[c].at[pl.ds(start, NPT)],
        )

    return body(x2, src2, dst2, zrows)


def _tc_linear(agg2, w0t, w1t, b2):
    BN = 2000

    def mm(a_ref, w0_ref, w1_ref, b_ref, o_ref):
        acc = jnp.dot(a_ref[0], w0_ref[...], preferred_element_type=jnp.float32)
        acc = acc + jnp.dot(a_ref[1], w1_ref[...], preferred_element_type=jnp.float32)
        o_ref[...] = acc + b_ref[...]

    return pl.pallas_call(
        mm,
        grid=(N_NODES // BN,),
        in_specs=[
            pl.BlockSpec((2, BN, DH), lambda i: (0, i, 0)),
            pl.BlockSpec((DH, D_OUT), lambda i: (0, 0)),
            pl.BlockSpec((DH, D_OUT), lambda i: (0, 0)),
            pl.BlockSpec((1, D_OUT), lambda i: (0, 0)),
        ],
        out_specs=pl.BlockSpec((BN, D_OUT), lambda i: (i, 0)),
        out_shape=jax.ShapeDtypeStruct((N_NODES, D_OUT), jnp.float32),
    )(agg2, w0t, w1t, b2)


def kernel(x, edge_index, W, b):
    src2 = edge_index[0].astype(jnp.int32).reshape(NS * NG, G, CH)
    dst2 = edge_index[1].astype(jnp.int32).reshape(NS * NG, G, CH)
    zrows = jnp.zeros((NPT, DH), jnp.float32)
    w0t = W[:, :DH].T
    w1t = W[:, DH:].T
    b2 = b.reshape(1, D_OUT)
    agg2 = _sc_aggregate(x, src2, dst2, zrows)
    return _tc_linear(agg2, w0t, w1t, b2)


# speedup vs baseline: 9.5254x; 1.0025x over previous
import functools

import jax
import jax.numpy as jnp
from jax import lax
from jax.experimental import pallas as pl
from jax.experimental.pallas import tpu as pltpu
from jax.experimental.pallas import tpu_sc as plsc

N_NODES = 10000
N_EDGES = 160000
D_IN = 256
D_OUT = 256
DH = D_IN // 2

NC = 2
NS = 16
CH = 80
G = 25
NG = N_EDGES // (G * CH * NS)
T = NG * G
R = 3
NPT = 632
LAST_START = N_NODES - NPT


def _sc_aggregate(x2, src2, dst2, zrows):
    mesh = plsc.VectorSubcoreMesh(core_axis_name="c", subcore_axis_name="s")

    @functools.partial(
        pl.kernel,
        out_type=jax.ShapeDtypeStruct((NC, N_NODES, DH), jnp.float32),
        mesh=mesh,
        scratch_types=[
            pltpu.VMEM_SHARED((N_NODES, DH), jnp.float32),
            pltpu.VMEM((2, G, CH), jnp.int32),
            pltpu.VMEM((2, G, CH), jnp.int32),
            pltpu.VMEM((R, CH, DH), jnp.float32),
            pltpu.SemaphoreType.DMA,
            pltpu.SemaphoreType.DMA,
        ],
    )
    def body(x2_hbm, src_hbm, dst_hbm, z_hbm, out_hbm, acc, sbuf, dbuf, rows,
             sem_g, sem_s):
        c = lax.axis_index("c")
        s = lax.axis_index("s")

        def gather(j):
            return pltpu.make_async_copy(
                x2_hbm.at[sbuf.at[(j // G) % 2, j % G], pl.ds(c * DH, DH)],
                rows.at[j % R], sem_g)

        def scatter(j):
            return pltpu.make_async_copy(
                rows.at[j % R],
                acc.at[dbuf.at[(j // G) % 2, j % G]], sem_s)

        def stage(g):
            pltpu.sync_copy(src_hbm.at[s * NG + g], sbuf.at[g % 2])
            pltpu.sync_copy(dst_hbm.at[s * NG + g], dbuf.at[g % 2])

        stage(0)
        for j0 in range(R - 1):
            gather(j0).start()
        start = pl.multiple_of(jnp.minimum(s * NPT, LAST_START), 8)
        pltpu.sync_copy(z_hbm, acc.at[pl.ds(start, NPT)])
        plsc.subcore_barrier()

        def step(j, _):
            @pl.when(j >= R - 1)
            def _():
                scatter(j - (R - 1)).wait()

            @pl.when(j + R - 1 < T)
            def _():
                jn = j + R - 1

                @pl.when(jn % G == 0)
                def _():
                    stage(jn // G)
                gather(jn).start()

            gather(j).wait()
            pltpu.async_copy(rows.at[j % R],
                             acc.at[dbuf.at[(j // G) % 2, j % G]],
                             sem_s, add=True)
            return 0

        lax.fori_loop(0, T, step, 0, unroll=False)
        for j0 in range(T - (R - 1), T):
            scatter(j0).wait()
        plsc.subcore_barrier()

        pltpu.sync_copy(
            acc.at[pl.ds(start, NPT)],
            out_hbm.at[c].at[pl.ds(start, NPT)],
        )

    return body(x2, src2, dst2, zrows)


def _tc_linear(agg2, W, b2):
    BN = 2000
    dn = (((1,), (1,)), ((), ()))

    def mm(a_ref, w_ref, b_ref, o_ref):
        acc = lax.dot_general(a_ref[0], w_ref[:, :DH], dn,
                              preferred_element_type=jnp.float32)
        acc = acc + lax.dot_general(a_ref[1], w_ref[:, DH:], dn,
                                    preferred_element_type=jnp.float32)
        o_ref[...] = acc + b_ref[...]

    return pl.pallas_call(
        mm,
        grid=(N_NODES // BN,),
        in_specs=[
            pl.BlockSpec((2, BN, DH), lambda i: (0, i, 0)),
            pl.BlockSpec((D_OUT, D_IN), lambda i: (0, 0)),
            pl.BlockSpec((1, D_OUT), lambda i: (0, 0)),
        ],
        out_specs=pl.BlockSpec((BN, D_OUT), lambda i: (i, 0)),
        out_shape=jax.ShapeDtypeStruct((N_NODES, D_OUT), jnp.float32),
    )(agg2, W, b2)


def kernel(x, edge_index, W, b):
    src2 = edge_index[0].astype(jnp.int32).reshape(NS * NG, G, CH)
    dst2 = edge_index[1].astype(jnp.int32).reshape(NS * NG, G, CH)
    zrows = jnp.zeros((NPT, DH), jnp.float32)
    b2 = b.reshape(1, D_OUT)
    agg2 = _sc_aggregate(x, src2, dst2, zrows)
    return _tc_linear(agg2, W, b2)


# speedup vs baseline: 9.6201x; 1.0099x over previous
import functools

import jax
import jax.numpy as jnp
from jax import lax
from jax.experimental import pallas as pl
from jax.experimental.pallas import tpu as pltpu
from jax.experimental.pallas import tpu_sc as plsc

N_NODES = 10000
N_EDGES = 160000
D_IN = 256
D_OUT = 256
DH = D_IN // 2

NC = 2
NS = 16
CH = 80
G = 5
NG = N_EDGES // (G * CH * NS)
T = NG * G
R = 3
NPT = 632
LAST_START = N_NODES - NPT


def _sc_aggregate(x, e4, zrows):
    mesh = plsc.VectorSubcoreMesh(core_axis_name="c", subcore_axis_name="s")

    @functools.partial(
        pl.kernel,
        out_type=jax.ShapeDtypeStruct((NC, N_NODES, DH), jnp.float32),
        mesh=mesh,
        scratch_types=[
            pltpu.VMEM_SHARED((N_NODES, DH), jnp.float32),
            pltpu.VMEM((2, 2, G, CH), jnp.int32),
            pltpu.VMEM((R, CH, DH), jnp.float32),
            pltpu.SemaphoreType.DMA,
            pltpu.SemaphoreType.DMA,
        ],
    )
    def body(x_hbm, e_hbm, z_hbm, out_hbm, acc, ibuf, rows, sem_g, sem_s):
        c = lax.axis_index("c")
        s = lax.axis_index("s")

        def gather(j):
            return pltpu.make_async_copy(
                x_hbm.at[ibuf.at[(j // G) % 2, 0, j % G], pl.ds(c * DH, DH)],
                rows.at[j % R], sem_g)

        def scatter(j):
            return pltpu.make_async_copy(
                rows.at[j % R],
                acc.at[ibuf.at[(j // G) % 2, 1, j % G]], sem_s)

        def stage(g):
            pltpu.sync_copy(e_hbm.at[s * NG + g], ibuf.at[g % 2])

        stage(0)
        for j0 in range(R - 1):
            gather(j0).start()
        start = pl.multiple_of(jnp.minimum(s * NPT, LAST_START), 8)
        pltpu.sync_copy(z_hbm, acc.at[pl.ds(start, NPT)])
        plsc.subcore_barrier()

        def step(j, _):
            @pl.when(j >= R - 1)
            def _():
                scatter(j - (R - 1)).wait()

            @pl.when(j + R - 1 < T)
            def _():
                jn = j + R - 1

                @pl.when(jn % G == 0)
                def _():
                    stage(jn // G)
                gather(jn).start()

            gather(j).wait()
            pltpu.async_copy(rows.at[j % R],
                             acc.at[ibuf.at[(j // G) % 2, 1, j % G]],
                             sem_s, add=True)
            return 0

        lax.fori_loop(0, T, step, 0, unroll=False)
        for j0 in range(T - (R - 1), T):
            scatter(j0).wait()
        plsc.subcore_barrier()

        pltpu.sync_copy(
            acc.at[pl.ds(start, NPT)],
            out_hbm.at[c].at[pl.ds(start, NPT)],
        )

    return body(x, e4, zrows)


def _tc_linear(agg2, W, b2):
    BN = 2000
    dn = (((1,), (1,)), ((), ()))

    def mm(a_ref, w_ref, b_ref, o_ref):
        acc = lax.dot_general(a_ref[0], w_ref[:, :DH], dn,
                              preferred_element_type=jnp.float32)
        acc = acc + lax.dot_general(a_ref[1], w_ref[:, DH:], dn,
                                    preferred_element_type=jnp.float32)
        o_ref[...] = acc + b_ref[...]

    return pl.pallas_call(
        mm,
        grid=(N_NODES // BN,),
        in_specs=[
            pl.BlockSpec((2, BN, DH), lambda i: (0, i, 0)),
            pl.BlockSpec((D_OUT, D_IN), lambda i: (0, 0)),
            pl.BlockSpec((1, D_OUT), lambda i: (0, 0)),
        ],
        out_specs=pl.BlockSpec((BN, D_OUT), lambda i: (i, 0)),
        out_shape=jax.ShapeDtypeStruct((N_NODES, D_OUT), jnp.float32),
    )(agg2, W, b2)


def kernel(x, edge_index, W, b):
    e4 = (edge_index.astype(jnp.int32)
          .reshape(2, NS * NG, G, CH).transpose(1, 0, 2, 3))
    zrows = jnp.zeros((NPT, DH), jnp.float32)
    b2 = b.reshape(1, D_OUT)
    agg2 = _sc_aggregate(x, e4, zrows)
    return _tc_linear(agg2, W, b2)
